# baseline (speedup 1.0000x reference)
import functools

import jax
import jax.numpy as jnp
from jax.experimental import pallas as pl

H = 128


def _obj_update_body(feat_ref, supp_ref, obj_ref, wv_ref, ws_ref, wo_ref,
                     bo_ref, w_obj_ref, dest_ref, wa_ref, ba_ref, wr_ref,
                     obj_out_ref, a_t_ref, r_ref):
    feat = feat_ref[...]
    obj = obj_ref[...]
    attn = (jnp.dot(feat, wv_ref[...], preferred_element_type=jnp.float32)
            + supp_ref[...] * ws_ref[...]
            + jnp.dot(obj, wo_ref[...], preferred_element_type=jnp.float32)
            + bo_ref[...])
    agg = jnp.sum(attn * feat, axis=0, keepdims=True)
    w_obj = w_obj_ref[...]
    obj_out_ref[...] = jax.nn.relu(
        jnp.dot(obj, w_obj[:H], preferred_element_type=jnp.float32)
        + jnp.dot(agg, w_obj[H:], preferred_element_type=jnp.float32))
    a_t_ref[...] = jax.lax.dot_general(
        wa_ref[...], feat, (((0,), (1,)), ((), ())),
        preferred_element_type=jnp.float32) + ba_ref[...]
    r_ref[...] = jnp.dot(dest_ref[...], wr_ref[...],
                         preferred_element_type=jnp.float32)


def _obj_update(feat, supp, obj, wv, ws, wo, bo, w_obj, dest, wa, ba, wr):
    n = feat.shape[0]
    nd = dest.shape[0]
    return pl.pallas_call(
        _obj_update_body,
        out_shape=(
            jax.ShapeDtypeStruct((1, H), jnp.float32),
            jax.ShapeDtypeStruct((1, n), jnp.float32),
            jax.ShapeDtypeStruct((nd, 1), jnp.float32),
        ),
    )(feat, supp, obj, wv, ws, wo, bo, w_obj, dest, wa, ba, wr)


def _flash_body(s_ref, feat_ref, a_t_ref, r_ref, sw_ref, obj_ref, dest_ref,
                wa_ref, wb_ref, out_ref, *, nj):
    j = pl.program_id(1)

    @pl.when(j == 0)
    def _():
        out_ref[...] = jnp.zeros_like(out_ref)

    s = s_ref[...]
    attn = jnp.where(s != 0.0,
                     s * sw_ref[...] + (a_t_ref[...] + r_ref[...]),
                     0.0)
    out_ref[...] += jnp.dot(attn, feat_ref[...],
                            preferred_element_type=jnp.float32)

    @pl.when(j == nj - 1)
    def _():
        wa = wa_ref[...]
        wb = wb_ref[...]
        oc = jax.nn.relu(
            jnp.dot(obj_ref[...], wa[:H], preferred_element_type=jnp.float32)
            + jnp.dot(dest_ref[...], wa[H:], preferred_element_type=jnp.float32))
        out_ref[...] = jax.nn.relu(
            jnp.dot(oc, wb[:H], preferred_element_type=jnp.float32)
            + jnp.dot(out_ref[...], wb[H:], preferred_element_type=jnp.float32))


def _flash_conv(s2d, feat, a_t, r, sw, obj, dest, wa, wb, bi=1024, bj=1024):
    ni_dim, nj_dim = s2d.shape
    ni, nj = ni_dim // bi, nj_dim // bj
    return pl.pallas_call(
        functools.partial(_flash_body, nj=nj),
        grid=(ni, nj),
        in_specs=[
            pl.BlockSpec((bi, bj), lambda i, j: (i, j)),
            pl.BlockSpec((bj, H), lambda i, j: (j, 0)),
            pl.BlockSpec((1, bj), lambda i, j: (0, j)),
            pl.BlockSpec((bi, 1), lambda i, j: (i, 0)),
            pl.BlockSpec((1, 1), lambda i, j: (0, 0)),
            pl.BlockSpec((1, H), lambda i, j: (0, 0)),
            pl.BlockSpec((bi, H), lambda i, j: (i, 0)),
            pl.BlockSpec((2 * H, H), lambda i, j: (0, 0)),
            pl.BlockSpec((2 * H, H), lambda i, j: (0, 0)),
        ],
        out_specs=pl.BlockSpec((bi, H), lambda i, j: (i, 0)),
        out_shape=jax.ShapeDtypeStruct((ni_dim, H), jnp.float32),
    )(s2d, feat, a_t, r, sw, obj, dest, wa, wb)


def kernel(col_hidden, row_hidden, obj_hidden, cv_supp, vc_supp, vo_supp,
           co_supp, vc_w, cv_w, co_w, oc_w, vo_w, ov_w,
           attn_vo_w, attn_vo_b, attn_cv_w, attn_cv_b,
           attn_co_w, attn_co_b, attn_vc_w, attn_vc_b):
    obj1, a1_t, r1 = _obj_update(
        col_hidden, vo_supp, obj_hidden,
        attn_vo_w[:H], attn_vo_w[H:H + 1], attn_vo_w[H + 1:],
        attn_vo_b.reshape(1, 1), vo_w,
        row_hidden, attn_cv_w[:H], attn_cv_b.reshape(1, 1), attn_cv_w[H + 1:])
    row_next = _flash_conv(cv_supp[0], col_hidden, a1_t, r1,
                           attn_cv_w[H:H + 1], obj1, row_hidden, oc_w, vc_w)
    obj2, a2_t, r2 = _obj_update(
        row_next, co_supp, obj1,
        attn_co_w[:H], attn_co_w[H:H + 1], attn_co_w[H + 1:],
        attn_co_b.reshape(1, 1), co_w,
        col_hidden, attn_vc_w[:H], attn_vc_b.reshape(1, 1), attn_vc_w[H + 1:])
    col_next = _flash_conv(vc_supp[0], row_next, a2_t, r2,
                           attn_vc_w[H:H + 1], obj2, col_hidden, ov_w, cv_w)
    return (col_next, row_next, obj2, cv_supp, vc_supp, vo_supp, co_supp)


# speedup vs baseline: 1.4559x; 1.4559x over previous
import functools

import jax
import jax.numpy as jnp
from jax.experimental import pallas as pl

H = 128


def _obj_update_body(feat_ref, supp_ref, obj_ref, wv_ref, ws_ref, wo_ref,
                     bo_ref, w_obj_ref, dest_ref, wa_ref, ba_ref, wr_ref,
                     obj_out_ref, a_t_ref, r_ref):
    feat = feat_ref[...]
    obj = obj_ref[...]
    attn = (jnp.dot(feat, wv_ref[...], preferred_element_type=jnp.float32)
            + supp_ref[...] * ws_ref[...]
            + jnp.dot(obj, wo_ref[...], preferred_element_type=jnp.float32)
            + bo_ref[...])
    agg = jnp.sum(attn * feat, axis=0, keepdims=True)
    w_obj = w_obj_ref[...]
    obj_out_ref[...] = jax.nn.relu(
        jnp.dot(obj, w_obj[:H], preferred_element_type=jnp.float32)
        + jnp.dot(agg, w_obj[H:], preferred_element_type=jnp.float32))
    a_t_ref[...] = jax.lax.dot_general(
        wa_ref[...], feat, (((0,), (1,)), ((), ())),
        preferred_element_type=jnp.float32) + ba_ref[...]
    r_ref[...] = jnp.dot(dest_ref[...], wr_ref[...],
                         preferred_element_type=jnp.float32)


def _obj_update(feat, supp, obj, wv, ws, wo, bo, w_obj, dest, wa, ba, wr):
    n = feat.shape[0]
    nd = dest.shape[0]
    return pl.pallas_call(
        _obj_update_body,
        out_shape=(
            jax.ShapeDtypeStruct((1, H), jnp.float32),
            jax.ShapeDtypeStruct((1, n), jnp.float32),
            jax.ShapeDtypeStruct((nd, 1), jnp.float32),
        ),
    )(feat, supp, obj, wv, ws, wo, bo, w_obj, dest, wa, ba, wr)


def _flash_body(s_ref, feat_ref, a_t_ref, r_ref, sw_ref, obj_ref, dest_ref,
                wa_ref, wb_ref, out_ref, s_out_ref):
    s = s_ref[...]
    s_out_ref[...] = s
    attn = jnp.where(s != 0.0,
                     s * sw_ref[...] + (a_t_ref[...] + r_ref[...]),
                     0.0)
    acc = jnp.dot(attn, feat_ref[...], preferred_element_type=jnp.float32)
    wa = wa_ref[...]
    wb = wb_ref[...]
    oc = jax.nn.relu(
        jnp.dot(obj_ref[...], wa[:H], preferred_element_type=jnp.float32)
        + jnp.dot(dest_ref[...], wa[H:], preferred_element_type=jnp.float32))
    out_ref[...] = jax.nn.relu(
        jnp.dot(oc, wb[:H], preferred_element_type=jnp.float32)
        + jnp.dot(acc, wb[H:], preferred_element_type=jnp.float32))


def _flash_conv(s2d, feat, a_t, r, sw, obj, dest, wa, wb, bi=512):
    ni_dim, n = s2d.shape
    ni = ni_dim // bi
    return pl.pallas_call(
        _flash_body,
        grid=(ni,),
        in_specs=[
            pl.BlockSpec((bi, n), lambda i: (i, 0)),
            pl.BlockSpec((n, H), lambda i: (0, 0)),
            pl.BlockSpec((1, n), lambda i: (0, 0)),
            pl.BlockSpec((bi, 1), lambda i: (i, 0)),
            pl.BlockSpec((1, 1), lambda i: (0, 0)),
            pl.BlockSpec((1, H), lambda i: (0, 0)),
            pl.BlockSpec((bi, H), lambda i: (i, 0)),
            pl.BlockSpec((2 * H, H), lambda i: (0, 0)),
            pl.BlockSpec((2 * H, H), lambda i: (0, 0)),
        ],
        out_specs=(
            pl.BlockSpec((bi, H), lambda i: (i, 0)),
            pl.BlockSpec((bi, n), lambda i: (i, 0)),
        ),
        out_shape=(
            jax.ShapeDtypeStruct((ni_dim, H), jnp.float32),
            jax.ShapeDtypeStruct((ni_dim, n), jnp.float32),
        ),
    )(s2d, feat, a_t, r, sw, obj, dest, wa, wb)


def kernel(col_hidden, row_hidden, obj_hidden, cv_supp, vc_supp, vo_supp,
           co_supp, vc_w, cv_w, co_w, oc_w, vo_w, ov_w,
           attn_vo_w, attn_vo_b, attn_cv_w, attn_cv_b,
           attn_co_w, attn_co_b, attn_vc_w, attn_vc_b):
    obj1, a1_t, r1 = _obj_update(
        col_hidden, vo_supp, obj_hidden,
        attn_vo_w[:H], attn_vo_w[H:H + 1], attn_vo_w[H + 1:],
        attn_vo_b.reshape(1, 1), vo_w,
        row_hidden, attn_cv_w[:H], attn_cv_b.reshape(1, 1), attn_cv_w[H + 1:])
    row_next, cv_copy = _flash_conv(cv_supp[0], col_hidden, a1_t, r1,
                                    attn_cv_w[H:H + 1], obj1, row_hidden,
                                    oc_w, vc_w)
    obj2, a2_t, r2 = _obj_update(
        row_next, co_supp, obj1,
        attn_co_w[:H], attn_co_w[H:H + 1], attn_co_w[H + 1:],
        attn_co_b.reshape(1, 1), co_w,
        col_hidden, attn_vc_w[:H], attn_vc_b.reshape(1, 1), attn_vc_w[H + 1:])
    col_next, vc_copy = _flash_conv(vc_supp[0], row_next, a2_t, r2,
                                    attn_vc_w[H:H + 1], obj2, col_hidden,
                                    ov_w, cv_w)
    return (col_next, row_next, obj2, cv_copy[None], vc_copy[None],
            vo_supp, co_supp)
